# baseline (speedup 1.0000x reference)
import jax
import jax.numpy as jnp
from jax.experimental import pallas as pl
from jax.experimental.pallas import tpu as pltpu

Q = 1024
D = 64
K = 100000
NL = 100
NN = 32
C = 2048
NCH = (K + C - 1) // C
KPAD = NCH * C
SUB = 1024
NBKT = 32
_BIG = 3.0e7
_EPS = 1.0 / NL


def _dot(a, b, dims):
    return jax.lax.dot_general(a, b, (dims, ((), ())),
                               preferred_element_type=jnp.float32,
                               precision=jax.lax.Precision.HIGHEST)


def _dotb(a, b, dims):
    return jax.lax.dot_general(a.astype(jnp.bfloat16), b.astype(jnp.bfloat16),
                               (dims, ((), ())),
                               preferred_element_type=jnp.float32)


def _body(xe_ref, xt_ref, m_ref, y_ref, o_ref,
          bmin_ref, yp_ref, tub_ref, topv_ref, topl_ref, dsub_ref):
    ph = pl.program_id(0)
    ch = pl.program_id(1)

    xe = xe_ref[...]
    xt = xt_ref[...]
    mm = m_ref[...]

    @pl.when((ph == 0) & (ch == 0))
    def _init():
        bmin_ref[...] = jnp.full((Q, 128), jnp.inf, jnp.float32)
        ym = _dotb(xe, mm, ((1,), (0,)))
        yp_ref[...] = jnp.sum(ym * xe, axis=1, keepdims=True)

    cache = _dotb(xt, mm, ((1,), (0,)))
    xp = _dot(jnp.ones((1, D), jnp.float32), cache * xt,
              ((1,), (1,)))
    cross = _dotb(xe, cache, ((1,), (1,)))
    d = (xp + yp_ref[...]) - 2.0 * cross

    li = jax.lax.broadcasted_iota(jnp.int32, (1, C), 1)
    valid = li < (K - ch * C)

    @pl.when(ph == 0)
    def _phase_a():
        da = jnp.where(valid, d, jnp.inf)
        b = bmin_ref[...]
        for g in range(C // 128):
            b = jnp.minimum(b, da[:, g * 128:(g + 1) * 128])
        bmin_ref[...] = b

    @pl.when((ph == 1) & (ch == 0))
    def _init_b():
        b = bmin_ref[...]
        f = jnp.minimum(jnp.minimum(b[:, 0:32], b[:, 32:64]),
                        jnp.minimum(b[:, 64:96], b[:, 96:128]))
        tub_ref[...] = jnp.max(f, axis=1, keepdims=True)
        topv_ref[...] = jnp.full((Q, NN), jnp.inf, jnp.float32)
        topl_ref[...] = jnp.zeros((Q, NN), jnp.float32)

    @pl.when(ph == 1)
    def _phase_b():
        tub = tub_ref[...]
        y = y_ref[...]
        dm = jnp.where(valid & (d <= tub), d, jnp.inf)
        lif = li.astype(jnp.float32)
        enc = lif * 128.0 + y
        nn_iota = jax.lax.broadcasted_iota(jnp.int32, (1, NN), 1).astype(jnp.float32)

        for s in range(C // SUB):
            sl = slice(s * SUB, (s + 1) * SUB)
            encs = enc[:, sl]
            lanes_s = lif[:, sl]
            ds0 = dm[:, sl]
            dsub_ref[...] = ds0
            nsurv = jnp.sum(jnp.where(ds0 < jnp.inf, 1.0, 0.0),
                            axis=1, keepdims=True)
            trip = jnp.minimum(jnp.max(nsurv), float(NN)).astype(jnp.int32)

            def _iter(i, c):
                del i
                ds2 = dsub_ref[...]
                m = jnp.min(ds2, axis=1, keepdims=True)
                ismin = ds2 == m
                e = jnp.min(jnp.where(ismin, encs, _BIG),
                            axis=1, keepdims=True)
                pos = jnp.floor(e * (1.0 / 128.0))
                lab = e - pos * 128.0
                sel = (lanes_s == pos) & ismin
                dsub_ref[...] = jnp.where(sel, jnp.inf, ds2)
                tv2 = topv_ref[...]
                tmax = jnp.max(tv2, axis=1, keepdims=True)
                ins = m < tmax
                ip = jnp.min(jnp.where(tv2 == tmax, nn_iota, _BIG),
                             axis=1, keepdims=True)
                seli = (nn_iota == ip) & ins
                topv_ref[...] = jnp.where(seli, m, tv2)
                topl_ref[...] = jnp.where(seli, lab, topl_ref[...])
                return c

            jax.lax.fori_loop(0, trip, _iter, 0)

    @pl.when((ph == 1) & (ch == NCH - 1))
    def _epilogue():
        tl = topl_ref[...]
        lab_iota = jax.lax.broadcasted_iota(jnp.int32, (1, NL), 1).astype(jnp.float32)
        acc = jnp.zeros((Q, NL), jnp.float32)
        for j in range(NN):
            acc = acc + jnp.where(tl[:, j:j + 1] == lab_iota, 1.0, 0.0)
        o_ref[...] = acc - lab_iota * _EPS


def kernel(X_eval, X_train, M, y_train):
    xt = jnp.pad(X_train, ((0, KPAD - K), (0, 0)))
    yf = jnp.pad(y_train.astype(jnp.float32), (0, KPAD - K)).reshape(1, KPAD)
    return pl.pallas_call(
        _body,
        grid=(2, NCH),
        in_specs=[
            pl.BlockSpec((Q, D), lambda p, c: (0, 0)),
            pl.BlockSpec((C, D), lambda p, c: (c, 0)),
            pl.BlockSpec((D, D), lambda p, c: (0, 0)),
            pl.BlockSpec((1, C), lambda p, c: (0, c)),
        ],
        out_specs=pl.BlockSpec((Q, NL), lambda p, c: (0, 0)),
        out_shape=jax.ShapeDtypeStruct((Q, NL), jnp.float32),
        scratch_shapes=[
            pltpu.VMEM((Q, 128), jnp.float32),
            pltpu.VMEM((Q, 1), jnp.float32),
            pltpu.VMEM((Q, 1), jnp.float32),
            pltpu.VMEM((Q, NN), jnp.float32),
            pltpu.VMEM((Q, NN), jnp.float32),
            pltpu.VMEM((Q, SUB), jnp.float32),
        ],
    )(X_eval, xt, M, yf)


# speedup vs baseline: 1.0442x; 1.0442x over previous
import jax
import jax.numpy as jnp
from jax.experimental import pallas as pl
from jax.experimental.pallas import tpu as pltpu

Q = 1024
D = 64
K = 100000
NL = 100
NN = 32
C = 4096
NCH = (K + C - 1) // C
KPAD = NCH * C
SUB = 512
NBKT = 32
_BIG = 3.0e7
_EPS = 1.0 / NL


def _dot(a, b, dims):
    return jax.lax.dot_general(a, b, (dims, ((), ())),
                               preferred_element_type=jnp.float32,
                               precision=jax.lax.Precision.HIGHEST)


def _dotb(a, b, dims):
    return jax.lax.dot_general(a.astype(jnp.bfloat16), b.astype(jnp.bfloat16),
                               (dims, ((), ())),
                               preferred_element_type=jnp.float32)


def _body(xe_ref, xt_ref, m_ref, y_ref, o_ref,
          bmin_ref, yp_ref, tub_ref, topv_ref, topl_ref, dsub_ref):
    ph = pl.program_id(0)
    ch = pl.program_id(1)

    xe = xe_ref[...]
    xt = xt_ref[...]
    mm = m_ref[...]

    @pl.when((ph == 0) & (ch == 0))
    def _init():
        bmin_ref[...] = jnp.full((Q, 128), jnp.inf, jnp.float32)
        ym = _dotb(xe, mm, ((1,), (0,)))
        yp_ref[...] = jnp.sum(ym * xe, axis=1, keepdims=True)

    cache = _dotb(xt, mm, ((1,), (0,)))
    xp = _dot(jnp.ones((1, D), jnp.float32), cache * xt,
              ((1,), (1,)))
    cross = _dotb(xe, cache, ((1,), (1,)))
    d = (xp + yp_ref[...]) - 2.0 * cross

    li = jax.lax.broadcasted_iota(jnp.int32, (1, C), 1)
    valid = li < (K - ch * C)

    @pl.when(ph == 0)
    def _phase_a():
        da = jnp.where(valid, d, jnp.inf)
        b = bmin_ref[...]
        for g in range(C // 128):
            b = jnp.minimum(b, da[:, g * 128:(g + 1) * 128])
        bmin_ref[...] = b

    @pl.when((ph == 1) & (ch == 0))
    def _init_b():
        b = bmin_ref[...]
        f = jnp.minimum(jnp.minimum(b[:, 0:32], b[:, 32:64]),
                        jnp.minimum(b[:, 64:96], b[:, 96:128]))
        tub_ref[...] = jnp.max(f, axis=1, keepdims=True)
        topv_ref[...] = jnp.full((Q, NN), jnp.inf, jnp.float32)
        topl_ref[...] = jnp.zeros((Q, NN), jnp.float32)

    @pl.when(ph == 1)
    def _phase_b():
        tub = tub_ref[...]
        y = y_ref[...]
        dm = jnp.where(valid & (d <= tub), d, jnp.inf)
        lif = li.astype(jnp.float32)
        enc = lif * 128.0 + y
        nn_iota = jax.lax.broadcasted_iota(jnp.int32, (1, NN), 1).astype(jnp.float32)

        for s in range(C // SUB):
            sl = slice(s * SUB, (s + 1) * SUB)
            encs = enc[:, sl]
            lanes_s = lif[:, sl]
            ds0 = dm[:, sl]
            dsub_ref[...] = ds0
            nsurv = jnp.sum(jnp.where(ds0 < jnp.inf, 1.0, 0.0),
                            axis=1, keepdims=True)
            trip = jnp.minimum(jnp.max(nsurv), float(NN)).astype(jnp.int32)

            def _iter(i, c):
                del i
                ds2 = dsub_ref[...]
                m = jnp.min(ds2, axis=1, keepdims=True)
                ismin = ds2 == m
                e = jnp.min(jnp.where(ismin, encs, _BIG),
                            axis=1, keepdims=True)
                pos = jnp.floor(e * (1.0 / 128.0))
                lab = e - pos * 128.0
                sel = (lanes_s == pos) & ismin
                dsub_ref[...] = jnp.where(sel, jnp.inf, ds2)
                tv2 = topv_ref[...]
                tmax = jnp.max(tv2, axis=1, keepdims=True)
                ins = m < tmax
                ip = jnp.min(jnp.where(tv2 == tmax, nn_iota, _BIG),
                             axis=1, keepdims=True)
                seli = (nn_iota == ip) & ins
                topv_ref[...] = jnp.where(seli, m, tv2)
                topl_ref[...] = jnp.where(seli, lab, topl_ref[...])
                return c

            jax.lax.fori_loop(0, trip, _iter, 0)

    @pl.when((ph == 1) & (ch == NCH - 1))
    def _epilogue():
        tl = topl_ref[...]
        lab_iota = jax.lax.broadcasted_iota(jnp.int32, (1, NL), 1).astype(jnp.float32)
        acc = jnp.zeros((Q, NL), jnp.float32)
        for j in range(NN):
            acc = acc + jnp.where(tl[:, j:j + 1] == lab_iota, 1.0, 0.0)
        o_ref[...] = acc - lab_iota * _EPS


def kernel(X_eval, X_train, M, y_train):
    xt = jnp.pad(X_train, ((0, KPAD - K), (0, 0)))
    yf = jnp.pad(y_train.astype(jnp.float32), (0, KPAD - K)).reshape(1, KPAD)
    return pl.pallas_call(
        _body,
        grid=(2, NCH),
        in_specs=[
            pl.BlockSpec((Q, D), lambda p, c: (0, 0)),
            pl.BlockSpec((C, D), lambda p, c: (c, 0)),
            pl.BlockSpec((D, D), lambda p, c: (0, 0)),
            pl.BlockSpec((1, C), lambda p, c: (0, c)),
        ],
        out_specs=pl.BlockSpec((Q, NL), lambda p, c: (0, 0)),
        out_shape=jax.ShapeDtypeStruct((Q, NL), jnp.float32),
        scratch_shapes=[
            pltpu.VMEM((Q, 128), jnp.float32),
            pltpu.VMEM((Q, 1), jnp.float32),
            pltpu.VMEM((Q, 1), jnp.float32),
            pltpu.VMEM((Q, NN), jnp.float32),
            pltpu.VMEM((Q, NN), jnp.float32),
            pltpu.VMEM((Q, SUB), jnp.float32),
        ],
    )(X_eval, xt, M, yf)


# speedup vs baseline: 1.0498x; 1.0054x over previous
import jax
import jax.numpy as jnp
from jax.experimental import pallas as pl
from jax.experimental.pallas import tpu as pltpu

Q = 1024
D = 64
K = 100000
NL = 100
NN = 32
C = 2048
NCH = (K + C - 1) // C
KPAD = NCH * C
SUB = 512
NBKT = 32
_BIG = 3.0e7
_EPS = 1.0 / NL


def _dot(a, b, dims):
    return jax.lax.dot_general(a, b, (dims, ((), ())),
                               preferred_element_type=jnp.float32,
                               precision=jax.lax.Precision.HIGHEST)


def _dotb(a, b, dims):
    return jax.lax.dot_general(a.astype(jnp.bfloat16), b.astype(jnp.bfloat16),
                               (dims, ((), ())),
                               preferred_element_type=jnp.float32)


def _body(xe_ref, xt_ref, m_ref, y_ref, o_ref,
          bmin_ref, yp_ref, tub_ref, topv_ref, topl_ref, dsub_ref):
    ph = pl.program_id(0)
    ch = pl.program_id(1)

    xe = xe_ref[...]
    xt = xt_ref[...]
    mm = m_ref[...]

    @pl.when((ph == 0) & (ch == 0))
    def _init():
        bmin_ref[...] = jnp.full((Q, 128), jnp.inf, jnp.float32)
        ym = _dotb(xe, mm, ((1,), (0,)))
        yp_ref[...] = jnp.sum(ym * xe, axis=1, keepdims=True)

    cache = _dotb(xt, mm, ((1,), (0,)))
    xp = _dot(jnp.ones((1, D), jnp.float32), cache * xt,
              ((1,), (1,)))
    cross = _dotb(xe, cache, ((1,), (1,)))
    d = (xp + yp_ref[...]) - 2.0 * cross

    li = jax.lax.broadcasted_iota(jnp.int32, (1, C), 1)
    valid = li < (K - ch * C)

    @pl.when(ph == 0)
    def _phase_a():
        da = jnp.where(valid, d, jnp.inf)
        b = bmin_ref[...]
        for g in range(C // 128):
            b = jnp.minimum(b, da[:, g * 128:(g + 1) * 128])
        bmin_ref[...] = b

    @pl.when((ph == 1) & (ch == 0))
    def _init_b():
        b = bmin_ref[...]
        f = jnp.minimum(jnp.minimum(b[:, 0:32], b[:, 32:64]),
                        jnp.minimum(b[:, 64:96], b[:, 96:128]))
        tub_ref[...] = jnp.max(f, axis=1, keepdims=True)
        topv_ref[...] = jnp.full((Q, NN), jnp.inf, jnp.float32)
        topl_ref[...] = jnp.zeros((Q, NN), jnp.float32)

    @pl.when(ph == 1)
    def _phase_b():
        tub = tub_ref[...]
        y = y_ref[...]
        dm = jnp.where(valid & (d <= tub), d, jnp.inf)
        lif = li.astype(jnp.float32)
        enc = lif * 128.0 + y
        nn_iota = jax.lax.broadcasted_iota(jnp.int32, (1, NN), 1).astype(jnp.float32)

        for s in range(C // SUB):
            sl = slice(s * SUB, (s + 1) * SUB)
            encs = enc[:, sl]
            lanes_s = lif[:, sl]
            ds0 = dm[:, sl]
            dsub_ref[...] = ds0
            nsurv = jnp.sum(jnp.where(ds0 < jnp.inf, 1.0, 0.0),
                            axis=1, keepdims=True)
            trip = jnp.minimum(jnp.max(nsurv), float(NN)).astype(jnp.int32)

            def _iter(i, c):
                del i
                ds2 = dsub_ref[...]
                m = jnp.min(ds2, axis=1, keepdims=True)
                ismin = ds2 == m
                e = jnp.min(jnp.where(ismin, encs, _BIG),
                            axis=1, keepdims=True)
                pos = jnp.floor(e * (1.0 / 128.0))
                lab = e - pos * 128.0
                sel = (lanes_s == pos) & ismin
                dsub_ref[...] = jnp.where(sel, jnp.inf, ds2)
                tv2 = topv_ref[...]
                tmax = jnp.max(tv2, axis=1, keepdims=True)
                ins = m < tmax
                ip = jnp.min(jnp.where(tv2 == tmax, nn_iota, _BIG),
                             axis=1, keepdims=True)
                seli = (nn_iota == ip) & ins
                topv_ref[...] = jnp.where(seli, m, tv2)
                topl_ref[...] = jnp.where(seli, lab, topl_ref[...])
                return c

            jax.lax.fori_loop(0, trip, _iter, 0)

    @pl.when((ph == 1) & (ch == NCH - 1))
    def _epilogue():
        tl = topl_ref[...]
        lab_iota = jax.lax.broadcasted_iota(jnp.int32, (1, NL), 1).astype(jnp.float32)
        acc = jnp.zeros((Q, NL), jnp.float32)
        for j in range(NN):
            acc = acc + jnp.where(tl[:, j:j + 1] == lab_iota, 1.0, 0.0)
        o_ref[...] = acc - lab_iota * _EPS


def kernel(X_eval, X_train, M, y_train):
    xt = jnp.pad(X_train, ((0, KPAD - K), (0, 0)))
    yf = jnp.pad(y_train.astype(jnp.float32), (0, KPAD - K)).reshape(1, KPAD)
    return pl.pallas_call(
        _body,
        grid=(2, NCH),
        in_specs=[
            pl.BlockSpec((Q, D), lambda p, c: (0, 0)),
            pl.BlockSpec((C, D), lambda p, c: (c, 0)),
            pl.BlockSpec((D, D), lambda p, c: (0, 0)),
            pl.BlockSpec((1, C), lambda p, c: (0, c)),
        ],
        out_specs=pl.BlockSpec((Q, NL), lambda p, c: (0, 0)),
        out_shape=jax.ShapeDtypeStruct((Q, NL), jnp.float32),
        scratch_shapes=[
            pltpu.VMEM((Q, 128), jnp.float32),
            pltpu.VMEM((Q, 1), jnp.float32),
            pltpu.VMEM((Q, 1), jnp.float32),
            pltpu.VMEM((Q, NN), jnp.float32),
            pltpu.VMEM((Q, NN), jnp.float32),
            pltpu.VMEM((Q, SUB), jnp.float32),
        ],
    )(X_eval, xt, M, yf)
